# baseline (speedup 1.0000x reference)
import jax
import jax.numpy as jnp
from jax import lax
from jax.experimental import pallas as pl
from jax.experimental.pallas import tpu as pltpu
from jax.experimental.pallas import tpu_sc as plsc

B = 16384
D = 64
NC, NS = 2, 16
NW = NC * NS
BPW = B // NW
BS = 1024


def _gather_body(idx_hbm, tab_hbm, out_hbm, idx_v, rows_v, sem):
    wid = lax.axis_index("s") * NC + lax.axis_index("c")
    base = wid * BPW
    pltpu.sync_copy(idx_hbm.at[pl.ds(base, BPW)], idx_v)
    pltpu.async_copy(tab_hbm.at[idx_v], rows_v, sem).wait()
    pltpu.sync_copy(rows_v, out_hbm.at[pl.ds(base, BPW)])


def _sc_gather_one(idx, tab):
    mesh = plsc.VectorSubcoreMesh(core_axis_name="c", subcore_axis_name="s")
    f = pl.kernel(
        _gather_body,
        mesh=mesh,
        compiler_params=pltpu.CompilerParams(use_tc_tiling_on_sc=False),
        out_type=jax.ShapeDtypeStruct((B, D), jnp.float32),
        scratch_types=[
            pltpu.VMEM((BPW,), jnp.int32),
            pltpu.VMEM((BPW, D), jnp.float32),
            pltpu.SemaphoreType.DMA,
        ],
    )
    return f(idx, tab)


def _mlp_body(u_ref, v_ref, w1_ref, b1_ref, w2_ref, b2_ref, w3_ref, b3_ref,
              o_ref):
    u = u_ref[...]
    v = v_ref[...]
    x = jnp.concatenate([u, v, u * v, jnp.abs(u - v)], axis=1)
    h = jnp.dot(x, w1_ref[...], preferred_element_type=jnp.float32) + b1_ref[...]
    h = jnp.maximum(h, 0.0)
    h = jnp.dot(h, w2_ref[...], preferred_element_type=jnp.float32) + b2_ref[...]
    h = jnp.maximum(h, 0.0)
    o_ref[...] = jnp.sum(h * w3_ref[...], axis=1) + b3_ref[0]


def _tc_mlp(u_g, v_g, w1t, b1, w2t, b2, w3, b3):
    grid = (B // BS,)
    return pl.pallas_call(
        _mlp_body,
        grid=grid,
        in_specs=[
            pl.BlockSpec((BS, D), lambda i: (i, 0)),
            pl.BlockSpec((BS, D), lambda i: (i, 0)),
            pl.BlockSpec((256, 256), lambda i: (0, 0)),
            pl.BlockSpec((1, 256), lambda i: (0, 0)),
            pl.BlockSpec((256, 64), lambda i: (0, 0)),
            pl.BlockSpec((1, 64), lambda i: (0, 0)),
            pl.BlockSpec((1, 64), lambda i: (0, 0)),
            pl.BlockSpec(memory_space=pltpu.SMEM),
        ],
        out_specs=pl.BlockSpec((BS,), lambda i: (i,)),
        out_shape=jax.ShapeDtypeStruct((B,), jnp.float32),
    )(u_g, v_g, w1t, b1, w2t, b2, w3, b3)


def kernel(user_idx, movie_idx, user_emb, item_emb, W1, b1, W2, b2, W3, b3):
    u_g = _sc_gather_one(user_idx, user_emb)
    v_g = _sc_gather_one(movie_idx, item_emb)
    return _tc_mlp(u_g, v_g, W1.T, b1.reshape(1, 256), W2.T, b2.reshape(1, 64),
                   W3, b3)


# speedup vs baseline: 1.0022x; 1.0022x over previous
import jax
import jax.numpy as jnp
from jax import lax
from jax.experimental import pallas as pl
from jax.experimental.pallas import tpu as pltpu
from jax.experimental.pallas import tpu_sc as plsc

B = 16384
D = 64
NC, NS = 2, 16
NW = NC * NS
BPW = B // NW
BS = 1024


def _gather_body(uidx_hbm, midx_hbm, utab_hbm, itab_hbm, uout_hbm, vout_hbm,
                 idx_v, idxh_v, rows_v, sem):
    wid = lax.axis_index("s") * NC + lax.axis_index("c")
    base = wid * BPW

    for (tab, idx_hbm, out) in ((utab_hbm, uidx_hbm, uout_hbm),
                                (itab_hbm, midx_hbm, vout_hbm)):
        pltpu.sync_copy(idx_hbm.at[pl.ds(base, BPW)], idx_v)

        def halve(g, _):
            idxh_v[pl.ds(g * 16, 16)] = idx_v[pl.ds(g * 16, 16)] >> 1
            return 0

        lax.fori_loop(0, BPW // 16, halve, 0)
        pltpu.async_copy(tab.at[idxh_v], rows_v, sem).wait()
        pltpu.sync_copy(rows_v, out.at[pl.ds(base, BPW)])


def _sc_gather(user_idx, movie_idx, utabP, itabP):
    mesh = plsc.VectorSubcoreMesh(core_axis_name="c", subcore_axis_name="s")
    f = pl.kernel(
        _gather_body,
        mesh=mesh,
        compiler_params=pltpu.CompilerParams(use_tc_tiling_on_sc=True),
        out_type=[
            jax.ShapeDtypeStruct((B, 2 * D), jnp.float32),
            jax.ShapeDtypeStruct((B, 2 * D), jnp.float32),
        ],
        scratch_types=[
            pltpu.VMEM((BPW,), jnp.int32),
            pltpu.VMEM((BPW,), jnp.int32),
            pltpu.VMEM((BPW, 2 * D), jnp.float32),
            pltpu.SemaphoreType.DMA,
        ],
    )
    return f(user_idx, movie_idx, utabP, itabP)


def _mlp_body(gu_ref, gv_ref, up_ref, vp_ref, w1_ref, b1_ref, w2_ref, b2_ref,
              w3_ref, b3_ref, o_ref):
    gu = gu_ref[...]
    gv = gv_ref[...]
    up = up_ref[...]
    vp = vp_ref[...]
    u = gu[:, :D] + (gu[:, D:] - gu[:, :D]) * up
    v = gv[:, :D] + (gv[:, D:] - gv[:, :D]) * vp
    x = jnp.concatenate([u, v, u * v, jnp.abs(u - v)], axis=1)
    h = jnp.dot(x, w1_ref[...], preferred_element_type=jnp.float32) + b1_ref[...]
    h = jnp.maximum(h, 0.0)
    h = jnp.dot(h, w2_ref[...], preferred_element_type=jnp.float32) + b2_ref[...]
    h = jnp.maximum(h, 0.0)
    o_ref[...] = jnp.sum(h * w3_ref[...], axis=1) + b3_ref[0]


def _tc_mlp(gu, gv, upar, vpar, w1t, b1, w2t, b2, w3, b3):
    grid = (B // BS,)
    return pl.pallas_call(
        _mlp_body,
        grid=grid,
        in_specs=[
            pl.BlockSpec((BS, 2 * D), lambda i: (i, 0)),
            pl.BlockSpec((BS, 2 * D), lambda i: (i, 0)),
            pl.BlockSpec((BS, 1), lambda i: (i, 0)),
            pl.BlockSpec((BS, 1), lambda i: (i, 0)),
            pl.BlockSpec((256, 256), lambda i: (0, 0)),
            pl.BlockSpec((1, 256), lambda i: (0, 0)),
            pl.BlockSpec((256, 64), lambda i: (0, 0)),
            pl.BlockSpec((1, 64), lambda i: (0, 0)),
            pl.BlockSpec((1, 64), lambda i: (0, 0)),
            pl.BlockSpec(memory_space=pltpu.SMEM),
        ],
        out_specs=pl.BlockSpec((BS,), lambda i: (i,)),
        out_shape=jax.ShapeDtypeStruct((B,), jnp.float32),
    )(gu, gv, upar, vpar, w1t, b1, w2t, b2, w3, b3)


def kernel(user_idx, movie_idx, user_emb, item_emb, W1, b1, W2, b2, W3, b3):
    utabP = user_emb.reshape(500000, 128)
    itabP = item_emb.reshape(500000, 128)
    gu, gv = _sc_gather(user_idx, movie_idx, utabP, itabP)
    upar = (user_idx & 1).astype(jnp.float32).reshape(B, 1)
    vpar = (movie_idx & 1).astype(jnp.float32).reshape(B, 1)
    return _tc_mlp(gu, gv, upar, vpar, W1.T, b1.reshape(1, 256),
                   W2.T, b2.reshape(1, 64), W3, b3)


# speedup vs baseline: 1.2712x; 1.2683x over previous
import jax
import jax.numpy as jnp
from jax import lax
from jax.experimental import pallas as pl
from jax.experimental.pallas import tpu as pltpu
from jax.experimental.pallas import tpu_sc as plsc

B = 16384
D = 64
NC, NS = 2, 16
NW = NC * NS
BPW = B // NW
BS = 1024
CH = 256


def _gather_user_body(idx_hbm, tab_hbm, out_hbm, idx_v, rows_v, sem):
    wid = lax.axis_index("s") * NC + lax.axis_index("c")
    base = wid * BPW
    pltpu.sync_copy(idx_hbm.at[pl.ds(base, BPW)], idx_v)
    lanes = lax.broadcasted_iota(jnp.int32, (16,), 0)

    for c in range(BPW // CH):
        off = c * CH

        def fire(k, _):
            vec = idx_v[pl.ds(off + k * 16, 16)]
            for l in range(16):
                i = jnp.sum(jnp.where(lanes == l, vec, 0))
                pltpu.async_copy(tab_hbm.at[pl.ds(i, 1), :],
                                 rows_v.at[pl.ds(k * 16 + l, 1), :], sem)
            return 0

        lax.fori_loop(0, CH // 16, fire, 0)

        def drain(j, _):
            pltpu.make_async_copy(tab_hbm.at[pl.ds(0, 1), :],
                                  rows_v.at[pl.ds(j, 1), :], sem).wait()
            return 0

        lax.fori_loop(0, CH, drain, 0)
        pltpu.sync_copy(rows_v, out_hbm.at[pl.ds(base + off, CH)])


def _sc_gather_user(user_idx, user_emb):
    mesh = plsc.VectorSubcoreMesh(core_axis_name="c", subcore_axis_name="s")
    f = pl.kernel(
        _gather_user_body,
        mesh=mesh,
        compiler_params=pltpu.CompilerParams(needs_layout_passes=False,
                                             use_tc_tiling_on_sc=True),
        out_type=jax.ShapeDtypeStruct((B, D), jnp.float32),
        scratch_types=[
            pltpu.VMEM((BPW,), jnp.int32),
            pltpu.VMEM((CH, D), jnp.float32),
            pltpu.SemaphoreType.DMA,
        ],
    )
    return f(user_idx, user_emb)


def _gather_item_body(idx_hbm, tab_hbm, out_hbm, idx_v, idxh_v, rows_v, sem):
    wid = lax.axis_index("s") * NC + lax.axis_index("c")
    base = wid * BPW
    pltpu.sync_copy(idx_hbm.at[pl.ds(base, BPW)], idx_v)

    def halve(g, _):
        idxh_v[pl.ds(g * 16, 16)] = idx_v[pl.ds(g * 16, 16)] >> 1
        return 0

    lax.fori_loop(0, BPW // 16, halve, 0)
    pltpu.async_copy(tab_hbm.at[idxh_v], rows_v, sem).wait()
    pltpu.sync_copy(rows_v, out_hbm.at[pl.ds(base, BPW)])


def _sc_gather_item(movie_idx, itabP):
    mesh = plsc.VectorSubcoreMesh(core_axis_name="c", subcore_axis_name="s")
    f = pl.kernel(
        _gather_item_body,
        mesh=mesh,
        compiler_params=pltpu.CompilerParams(use_tc_tiling_on_sc=True),
        out_type=jax.ShapeDtypeStruct((B, 2 * D), jnp.float32),
        scratch_types=[
            pltpu.VMEM((BPW,), jnp.int32),
            pltpu.VMEM((BPW,), jnp.int32),
            pltpu.VMEM((BPW, 2 * D), jnp.float32),
            pltpu.SemaphoreType.DMA,
        ],
    )
    return f(movie_idx, itabP)


def _mlp_body(u_ref, gv_ref, vp_ref, w1_ref, b1_ref, w2_ref, b2_ref, w3_ref,
              b3_ref, o_ref):
    u = u_ref[...]
    gv = gv_ref[...]
    vp = vp_ref[...]
    v = gv[:, :D] + (gv[:, D:] - gv[:, :D]) * vp
    x = jnp.concatenate([u, v, u * v, jnp.abs(u - v)], axis=1)
    h = jnp.dot(x, w1_ref[...], preferred_element_type=jnp.float32) + b1_ref[...]
    h = jnp.maximum(h, 0.0)
    h = jnp.dot(h, w2_ref[...], preferred_element_type=jnp.float32) + b2_ref[...]
    h = jnp.maximum(h, 0.0)
    o_ref[...] = jnp.sum(h * w3_ref[...], axis=1) + b3_ref[0]


def _tc_mlp(u_g, gv, vpar, w1t, b1, w2t, b2, w3, b3):
    grid = (B // BS,)
    return pl.pallas_call(
        _mlp_body,
        grid=grid,
        in_specs=[
            pl.BlockSpec((BS, D), lambda i: (i, 0)),
            pl.BlockSpec((BS, 2 * D), lambda i: (i, 0)),
            pl.BlockSpec((BS, 1), lambda i: (i, 0)),
            pl.BlockSpec((256, 256), lambda i: (0, 0)),
            pl.BlockSpec((1, 256), lambda i: (0, 0)),
            pl.BlockSpec((256, 64), lambda i: (0, 0)),
            pl.BlockSpec((1, 64), lambda i: (0, 0)),
            pl.BlockSpec((1, 64), lambda i: (0, 0)),
            pl.BlockSpec(memory_space=pltpu.SMEM),
        ],
        out_specs=pl.BlockSpec((BS,), lambda i: (i,)),
        out_shape=jax.ShapeDtypeStruct((B,), jnp.float32),
    )(u_g, gv, vpar, w1t, b1, w2t, b2, w3, b3)


def kernel(user_idx, movie_idx, user_emb, item_emb, W1, b1, W2, b2, W3, b3):
    u_g = _sc_gather_user(user_idx, user_emb)
    gv = _sc_gather_item(movie_idx, item_emb.reshape(500000, 128))
    vpar = (movie_idx & 1).astype(jnp.float32).reshape(B, 1)
    return _tc_mlp(u_g, gv, vpar, W1.T, b1.reshape(1, 256),
                   W2.T, b2.reshape(1, 64), W3, b3)


# speedup vs baseline: 1.5608x; 1.2278x over previous
import jax
import jax.numpy as jnp
from jax import lax
from jax.experimental import pallas as pl
from jax.experimental.pallas import tpu as pltpu
from jax.experimental.pallas import tpu_sc as plsc

B = 16384
D = 64
NC, NS = 2, 16
NW = NC * NS
BPW = B // NW
BS = 1024
CH = 256


def _gather_body(uidx_hbm, midx_hbm, utab_hbm, itab_hbm, uout_hbm, vout_hbm,
                 uidx_v, midx_v, urows_v, vrows_v, sem_u, sem_v):
    wid = lax.axis_index("s") * NC + lax.axis_index("c")
    base = wid * BPW
    pltpu.sync_copy(uidx_hbm.at[pl.ds(base, BPW)], uidx_v)
    pltpu.sync_copy(midx_hbm.at[pl.ds(base, BPW)], midx_v)
    lanes = lax.broadcasted_iota(jnp.int32, (16,), 0)

    for c in range(BPW // CH):
        off = c * CH

        def fire(k, _):
            vec_u = uidx_v[pl.ds(off + k * 16, 16)]
            vec_v = midx_v[pl.ds(off + k * 16, 16)]
            for l in range(16):
                iu = jnp.sum(jnp.where(lanes == l, vec_u, 0))
                iv = jnp.sum(jnp.where(lanes == l, vec_v, 0))
                pltpu.async_copy(utab_hbm.at[pl.ds(iu, 1), :],
                                 urows_v.at[pl.ds(k * 16 + l, 1), :], sem_u)
                pltpu.async_copy(itab_hbm.at[pl.ds(iv, 1), :],
                                 vrows_v.at[pl.ds(k * 16 + l, 1), :], sem_v)
            return 0

        lax.fori_loop(0, CH // 16, fire, 0)

        def drain(j, _):
            pltpu.make_async_copy(utab_hbm.at[pl.ds(0, 1), :],
                                  urows_v.at[pl.ds(j, 1), :], sem_u).wait()
            pltpu.make_async_copy(itab_hbm.at[pl.ds(0, 1), :],
                                  vrows_v.at[pl.ds(j, 1), :], sem_v).wait()
            return 0

        lax.fori_loop(0, CH, drain, 0)
        pltpu.sync_copy(urows_v, uout_hbm.at[pl.ds(base + off, CH)])
        pltpu.sync_copy(vrows_v, vout_hbm.at[pl.ds(base + off, CH)])


def _sc_gather(user_idx, movie_idx, user_emb, item_emb):
    mesh = plsc.VectorSubcoreMesh(core_axis_name="c", subcore_axis_name="s")
    f = pl.kernel(
        _gather_body,
        mesh=mesh,
        compiler_params=pltpu.CompilerParams(needs_layout_passes=False,
                                             use_tc_tiling_on_sc=True),
        out_type=[
            jax.ShapeDtypeStruct((B, D), jnp.float32),
            jax.ShapeDtypeStruct((B, D), jnp.float32),
        ],
        scratch_types=[
            pltpu.VMEM((BPW,), jnp.int32),
            pltpu.VMEM((BPW,), jnp.int32),
            pltpu.VMEM((CH, D), jnp.float32),
            pltpu.VMEM((CH, D), jnp.float32),
            pltpu.SemaphoreType.DMA,
            pltpu.SemaphoreType.DMA,
        ],
    )
    return f(user_idx, movie_idx, user_emb, item_emb)


def _mlp_body(u_ref, v_ref, w1_ref, b1_ref, w2_ref, b2_ref, w3_ref, b3_ref,
              o_ref):
    u = u_ref[...]
    v = v_ref[...]
    x = jnp.concatenate([u, v, u * v, jnp.abs(u - v)], axis=1)
    h = jnp.dot(x, w1_ref[...], preferred_element_type=jnp.float32) + b1_ref[...]
    h = jnp.maximum(h, 0.0)
    h = jnp.dot(h, w2_ref[...], preferred_element_type=jnp.float32) + b2_ref[...]
    h = jnp.maximum(h, 0.0)
    o_ref[...] = jnp.sum(h * w3_ref[...], axis=1) + b3_ref[0]


def _tc_mlp(u_g, v_g, w1t, b1, w2t, b2, w3, b3):
    grid = (B // BS,)
    return pl.pallas_call(
        _mlp_body,
        grid=grid,
        in_specs=[
            pl.BlockSpec((BS, D), lambda i: (i, 0)),
            pl.BlockSpec((BS, D), lambda i: (i, 0)),
            pl.BlockSpec((256, 256), lambda i: (0, 0)),
            pl.BlockSpec((1, 256), lambda i: (0, 0)),
            pl.BlockSpec((256, 64), lambda i: (0, 0)),
            pl.BlockSpec((1, 64), lambda i: (0, 0)),
            pl.BlockSpec((1, 64), lambda i: (0, 0)),
            pl.BlockSpec(memory_space=pltpu.SMEM),
        ],
        out_specs=pl.BlockSpec((BS,), lambda i: (i,)),
        out_shape=jax.ShapeDtypeStruct((B,), jnp.float32),
    )(u_g, v_g, w1t, b1, w2t, b2, w3, b3)


def kernel(user_idx, movie_idx, user_emb, item_emb, W1, b1, W2, b2, W3, b3):
    u_g, v_g = _sc_gather(user_idx, movie_idx, user_emb, item_emb)
    return _tc_mlp(u_g, v_g, W1.T, b1.reshape(1, 256), W2.T, b2.reshape(1, 64),
                   W3, b3)
